# Initial kernel scaffold; baseline (speedup 1.0000x reference)
import jax
import jax.numpy as jnp
from jax.experimental import pallas as pl


def kernel(x, T1, T2):
    raise NotImplementedError("write your pallas kernel here")


# speedup vs baseline: 1.3361x; 1.3361x over previous
import functools

import jax
import jax.numpy as jnp
from jax import lax
from jax.experimental import pallas as pl
from jax.experimental.pallas import tpu as pltpu
from jax.experimental.pallas import tpu_sc as plsc


@functools.lru_cache(maxsize=None)
def _build(B, V, D1, D2):
    NC, NS = 2, 16
    NW = NC * NS
    b_per_w = B // NW
    C = 64
    NCH = b_per_w // C

    mesh = plsc.VectorSubcoreMesh(core_axis_name="c", subcore_axis_name="s")

    @functools.partial(
        pl.kernel,
        out_type=(
            jax.ShapeDtypeStruct((B, D1), jnp.float32),
            jax.ShapeDtypeStruct((B, D2), jnp.float32),
        ),
        mesh=mesh,
        scratch_types=[
            pltpu.VMEM((b_per_w,), jnp.int32),
            pltpu.VMEM((C, D1), jnp.float32),
            pltpu.VMEM((C, D2), jnp.float32),
            pltpu.SemaphoreType.DMA,
            pltpu.SemaphoreType.DMA,
        ],
    )
    def k(x_hbm, t1_hbm, t2_hbm, o1_hbm, o2_hbm, idx_v, b1, b2, s1, s2):
        wid = lax.axis_index("s") * NC + lax.axis_index("c")
        base = wid * b_per_w
        pltpu.sync_copy(x_hbm.at[pl.ds(base, b_per_w)], idx_v)

        def body(g, carry):
            isl = idx_v.at[pl.ds(g * C, C)]
            cp1 = pltpu.async_copy(t1_hbm.at[isl], b1, s1)
            cp2 = pltpu.async_copy(t2_hbm.at[isl], b2, s2)
            cp1.wait()
            cp2.wait()
            pltpu.sync_copy(b1, o1_hbm.at[pl.ds(base + g * C, C)])
            pltpu.sync_copy(b2, o2_hbm.at[pl.ds(base + g * C, C)])
            return carry

        lax.fori_loop(0, NCH, body, 0)

    return k


def kernel(x, T1, T2):
    B = x.shape[0]
    V, d1 = T1.shape[0], T1.shape[1]
    d2 = T2.shape[1]
    D1, D2 = d1 * d1, d2 * d2
    t1 = T1.reshape(V, D1)
    t2 = T2.reshape(V, D2)
    xi = x.astype(jnp.int32)
    o1, o2 = _build(B, V, D1, D2)(xi, t1, t2)
    return o1.reshape(B * d1, d1), o2.reshape(B * d2, d2)


# speedup vs baseline: 1.3512x; 1.0113x over previous
import functools

import jax
import jax.numpy as jnp
from jax import lax
from jax.experimental import pallas as pl
from jax.experimental.pallas import tpu as pltpu
from jax.experimental.pallas import tpu_sc as plsc


@functools.lru_cache(maxsize=None)
def _build(B, V, D1, D2):
    NC, NS = 2, 16
    NW = NC * NS
    b_per_w = B // NW
    C = 32
    NCH = b_per_w // C

    mesh = plsc.VectorSubcoreMesh(core_axis_name="c", subcore_axis_name="s")

    @functools.partial(
        pl.kernel,
        out_type=(
            jax.ShapeDtypeStruct((B, D1), jnp.float32),
            jax.ShapeDtypeStruct((B, D2), jnp.float32),
        ),
        mesh=mesh,
        scratch_types=[
            pltpu.VMEM((b_per_w,), jnp.int32),
            pltpu.VMEM((2, C, D1), jnp.float32),
            pltpu.VMEM((2, C, D2), jnp.float32),
            pltpu.SemaphoreType.DMA((2,)),
            pltpu.SemaphoreType.DMA((2,)),
        ],
    )
    def k(x_hbm, t1_hbm, t2_hbm, o1_hbm, o2_hbm, idx_v, b1, b2, s1, s2):
        wid = lax.axis_index("s") * NC + lax.axis_index("c")
        base = wid * b_per_w
        pltpu.sync_copy(x_hbm.at[pl.ds(base, b_per_w)], idx_v)

        def start(g, slot):
            isl = idx_v.at[pl.ds(g * C, C)]
            pltpu.async_copy(t1_hbm.at[isl], b1.at[slot], s1.at[slot])
            pltpu.async_copy(t2_hbm.at[isl], b2.at[slot], s2.at[slot])

        def finish(g, slot):
            isl = idx_v.at[pl.ds(g * C, C)]
            pltpu.make_async_copy(t1_hbm.at[isl], b1.at[slot], s1.at[slot]).wait()
            pltpu.make_async_copy(t2_hbm.at[isl], b2.at[slot], s2.at[slot]).wait()
            pltpu.sync_copy(b1.at[slot], o1_hbm.at[pl.ds(base + g * C, C)])
            pltpu.sync_copy(b2.at[slot], o2_hbm.at[pl.ds(base + g * C, C)])

        start(0, 0)
        start(1, 1)

        def body(h, carry):
            g0 = 2 * h
            for b in range(2):
                g = g0 + b
                finish(g, b)
                pl.when(g + 2 < NCH)(lambda: start(g + 2, b))
            return carry

        lax.fori_loop(0, NCH // 2, body, 0)

    return k


def kernel(x, T1, T2):
    B = x.shape[0]
    V, d1 = T1.shape[0], T1.shape[1]
    d2 = T2.shape[1]
    D1, D2 = d1 * d1, d2 * d2
    t1 = T1.reshape(V, D1)
    t2 = T2.reshape(V, D2)
    xi = x.astype(jnp.int32)
    o1, o2 = _build(B, V, D1, D2)(xi, t1, t2)
    return o1.reshape(B * d1, d1), o2.reshape(B * d2, d2)
